# Initial kernel scaffold; baseline (speedup 1.0000x reference)
import functools
import jax
import jax.numpy as jnp
from jax import lax
from jax.experimental import pallas as pl
from jax.experimental.pallas import tpu as pltpu
from jax.experimental.pallas import tpu_sc as plsc

NN = 10000
NP = 10240
EE = 320000
EP = 327680
DI = 128
HH = 32
CC = 10
GG = 64

NC = 2
NS = 16
NW = NC * NS
CHUNK = 128
CPW = EP // NW // CHUNK
RPS = NP // NS

_SC_MESH = plsc.VectorSubcoreMesh(core_axis_name="c", subcore_axis_name="s")


def _sc_deg_body(dst_hbm, zeros_hbm, deg_hbm, idx_dst, ones_v, deg_sp, sem):
    c = lax.axis_index("c")
    s = lax.axis_index("s")
    wid = s * NC + c
    pltpu.sync_copy(zeros_hbm.at[pl.ds(s * RPS, RPS)],
                    deg_sp.at[pl.ds(s * RPS, RPS)])
    pltpu.async_copy(dst_hbm.at[wid], idx_dst, sem).wait()
    for i in range(CHUNK // 16):
        ones_v[pl.ds(i * 16, 16)] = jnp.ones((16,), jnp.float32)
    plsc.subcore_barrier()

    def body(j, carry):
        pltpu.sync_copy(ones_v, deg_sp.at[idx_dst.at[j]], add=True)
        return carry

    lax.fori_loop(0, CPW, body, 0)
    plsc.subcore_barrier()
    pltpu.sync_copy(deg_sp.at[pl.ds(s * RPS, RPS)],
                    deg_hbm.at[c, pl.ds(s * RPS, RPS)])


_sc_deg = functools.partial(
    pl.kernel,
    out_type=jax.ShapeDtypeStruct((NC, NP), jnp.float32),
    mesh=_SC_MESH,
    scratch_types=[
        pltpu.VMEM((CPW, CHUNK), jnp.int32),
        pltpu.VMEM((CHUNK,), jnp.float32),
        pltpu.VMEM_SHARED((NP,), jnp.float32),
        pltpu.SemaphoreType.DMA,
    ],
)(_sc_deg_body)


def _sc_edge_body(src_hbm, dst_hbm, g_hbm, zeros_hbm, acc_hbm,
                  idx_src, idx_dst, rows, acc_sp, sem):
    c = lax.axis_index("c")
    s = lax.axis_index("s")
    wid = s * NC + c
    pltpu.sync_copy(zeros_hbm.at[pl.ds(s * RPS, RPS)],
                    acc_sp.at[pl.ds(s * RPS, RPS)])
    pltpu.async_copy(src_hbm.at[wid], idx_src, sem).wait()
    pltpu.async_copy(dst_hbm.at[wid], idx_dst, sem).wait()
    plsc.subcore_barrier()

    def body(j, carry):
        pltpu.async_copy(g_hbm.at[idx_src.at[j]], rows, sem).wait()
        pltpu.sync_copy(rows, acc_sp.at[idx_dst.at[j]], add=True)
        return carry

    lax.fori_loop(0, CPW, body, 0)
    plsc.subcore_barrier()
    pltpu.sync_copy(acc_sp.at[pl.ds(s * RPS, RPS)],
                    acc_hbm.at[c, pl.ds(s * RPS, RPS)])


_sc_edge = functools.partial(
    pl.kernel,
    out_type=jax.ShapeDtypeStruct((NC, NP, HH), jnp.float32),
    mesh=_SC_MESH,
    scratch_types=[
        pltpu.VMEM((CPW, CHUNK), jnp.int32),
        pltpu.VMEM((CPW, CHUNK), jnp.int32),
        pltpu.VMEM((CHUNK, HH), jnp.float32),
        pltpu.VMEM_SHARED((NP, HH), jnp.float32),
        pltpu.SemaphoreType.DMA,
    ],
)(_sc_edge_body)


def _tc_dinv_body(deg_ref, out_ref):
    d = deg_ref[0] + deg_ref[1] + 1.0
    out_ref[...] = lax.rsqrt(d)


def _tc_dinv(deg2d):
    return pl.pallas_call(
        _tc_dinv_body,
        out_shape=jax.ShapeDtypeStruct((NP // 128, 128), jnp.float32),
    )(deg2d)


def _tc_pre_body(x_ref, w_ref, dinv_ref, g_ref):
    h = jnp.dot(x_ref[...], w_ref[...], preferred_element_type=jnp.float32)
    g_ref[...] = dinv_ref[...] * h


def _tc_pre(xp, W, dinv32):
    blk = 1024
    return pl.pallas_call(
        _tc_pre_body,
        grid=(NP // blk,),
        in_specs=[
            pl.BlockSpec((blk, DI), lambda i: (i, 0)),
            pl.BlockSpec((DI, HH), lambda i: (0, 0)),
            pl.BlockSpec((blk, HH), lambda i: (i, 0)),
        ],
        out_specs=pl.BlockSpec((blk, HH), lambda i: (i, 0)),
        out_shape=jax.ShapeDtypeStruct((NP, HH), jnp.float32),
    )(xp, W, dinv32)


def _norm_relu(acc_ref, g_ref, dinv_ref, b_ref):
    s = acc_ref[0] + acc_ref[1] + g_ref[...]
    pre = dinv_ref[...] * s + b_ref[...]
    nrm = jnp.sqrt(jnp.sum(pre * pre, axis=1, keepdims=True))
    return jnp.maximum(pre / jnp.maximum(nrm, 1e-12), 0.0)


def _tc_post_body(acc_ref, g_ref, dinv_ref, b_ref, w_ref, gnext_ref):
    o = _norm_relu(acc_ref, g_ref, dinv_ref, b_ref)
    gnext_ref[...] = dinv_ref[...] * jnp.dot(
        o, w_ref[...], preferred_element_type=jnp.float32)


def _tc_post(acc, g, dinv32, b, Wn):
    blk = 1024
    return pl.pallas_call(
        _tc_post_body,
        grid=(NP // blk,),
        in_specs=[
            pl.BlockSpec((NC, blk, HH), lambda i: (0, i, 0)),
            pl.BlockSpec((blk, HH), lambda i: (i, 0)),
            pl.BlockSpec((blk, HH), lambda i: (i, 0)),
            pl.BlockSpec((1, HH), lambda i: (0, 0)),
            pl.BlockSpec((HH, HH), lambda i: (0, 0)),
        ],
        out_specs=pl.BlockSpec((blk, HH), lambda i: (i, 0)),
        out_shape=jax.ShapeDtypeStruct((NP, HH), jnp.float32),
    )(acc, g, dinv32, b, Wn)


def _tc_readout_body(acc_ref, g_ref, dinv_ref, b_ref, batch_ref,
                     linw_ref, linb_ref, out_ref, mx_s, sm_s, ct_s):
    i = pl.program_id(0)
    nblk = pl.num_programs(0)

    @pl.when(i == 0)
    def _init():
        mx_s[...] = jnp.zeros_like(mx_s)
        sm_s[...] = jnp.zeros_like(sm_s)
        ct_s[...] = jnp.zeros_like(ct_s)

    o = _norm_relu(acc_ref, g_ref, dinv_ref, b_ref)
    batch = batch_ref[...]
    lo = batch_ref[0, 0]
    hi = batch_ref[batch_ref.shape[0] - 1, 0]

    def body(gid, carry):
        m = batch == gid
        vals = jnp.where(m, o, 0.0)
        mx = jnp.max(vals, axis=0, keepdims=True)
        sm = jnp.sum(vals, axis=0, keepdims=True)
        ct = jnp.sum(jnp.where(m, 1.0, 0.0), axis=0, keepdims=True)
        mx_s[pl.ds(gid, 1), :] = jnp.maximum(mx_s[pl.ds(gid, 1), :], mx)
        sm_s[pl.ds(gid, 1), :] = sm_s[pl.ds(gid, 1), :] + sm
        ct_s[pl.ds(gid, 1), :] = ct_s[pl.ds(gid, 1), :] + ct
        return carry

    lax.fori_loop(lo, hi + 1, body, 0)

    @pl.when(i == nblk - 1)
    def _fin():
        mean = sm_s[0:GG, :] / jnp.maximum(ct_s[0:GG, :], 1.0)
        pooled = jnp.concatenate([mx_s[0:GG, :], mean], axis=1)
        out_ref[...] = jnp.dot(
            pooled, linw_ref[...], preferred_element_type=jnp.float32
        ) + linb_ref[...]


def _tc_readout(acc, g, dinv32, b, batchb, linW, linb):
    blk = 512
    return pl.pallas_call(
        _tc_readout_body,
        grid=(NP // blk,),
        in_specs=[
            pl.BlockSpec((NC, blk, HH), lambda i: (0, i, 0)),
            pl.BlockSpec((blk, HH), lambda i: (i, 0)),
            pl.BlockSpec((blk, HH), lambda i: (i, 0)),
            pl.BlockSpec((1, HH), lambda i: (0, 0)),
            pl.BlockSpec((blk, HH), lambda i: (i, 0)),
            pl.BlockSpec((2 * HH, CC), lambda i: (0, 0)),
            pl.BlockSpec((1, CC), lambda i: (0, 0)),
        ],
        out_specs=pl.BlockSpec((GG, CC), lambda i: (0, 0)),
        out_shape=jax.ShapeDtypeStruct((GG, CC), jnp.float32),
        scratch_shapes=[
            pltpu.VMEM((GG + 8, HH), jnp.float32),
            pltpu.VMEM((GG + 8, HH), jnp.float32),
            pltpu.VMEM((GG + 8, HH), jnp.float32),
        ],
    )(acc, g, dinv32, b, batchb, linW, linb)


def kernel(x, edge_index, batch, W1, b1, W2, b2, W3, b3, linW, linb):
    f32 = jnp.float32
    xp = jnp.zeros((NP, DI), f32).at[:NN].set(x)
    pad_ids = (NN + jnp.arange(EP - EE, dtype=jnp.int32) % (NP - NN))
    src = jnp.concatenate([edge_index[0], pad_ids]).reshape(NW, CPW, CHUNK)
    dst = jnp.concatenate([edge_index[1], pad_ids]).reshape(NW, CPW, CHUNK)
    batchp = jnp.full((NP,), GG, jnp.int32).at[:NN].set(batch)
    batchb = jnp.broadcast_to(batchp[:, None], (NP, HH))
    zeros_nh = jnp.zeros((NP, HH), f32)
    zeros_n = jnp.zeros((NP,), f32)

    deg2 = _sc_deg(dst, zeros_n)
    dinv2d = _tc_dinv(deg2.reshape(NC, NP // 128, 128))
    dinv32 = jnp.broadcast_to(dinv2d.reshape(NP, 1), (NP, HH))

    g1 = _tc_pre(xp, W1, dinv32)
    acc1 = _sc_edge(src, dst, g1, zeros_nh)
    g2 = _tc_post(acc1, g1, dinv32, b1.reshape(1, HH), W2)
    acc2 = _sc_edge(src, dst, g2, zeros_nh)
    g3 = _tc_post(acc2, g2, dinv32, b2.reshape(1, HH), W3)
    acc3 = _sc_edge(src, dst, g3, zeros_nh)
    return _tc_readout(acc3, g3, dinv32, b3.reshape(1, HH), batchb,
                       linW, linb.reshape(1, CC))


# speedup vs baseline: 28.1511x; 28.1511x over previous
import functools
import jax
import jax.numpy as jnp
from jax import lax
from jax.experimental import pallas as pl
from jax.experimental.pallas import tpu as pltpu
from jax.experimental.pallas import tpu_sc as plsc

NN = 10000
NP = 10240
EE = 320000
EP = 327680
DI = 128
HH = 32
CC = 10
GG = 64

NC = 2
NS = 16
NW = NC * NS
CHUNK = 128
CPW = EP // NW // CHUNK
RPS = NP // NS

_SC_MESH = plsc.VectorSubcoreMesh(core_axis_name="c", subcore_axis_name="s")
_SC_PARAMS = pltpu.CompilerParams(use_tc_tiling_on_sc=False)


def _sc_deg_body(dst_hbm, zeros_hbm, deg_hbm, idx_dst, ones_v, deg_sp, sem):
    c = lax.axis_index("c")
    s = lax.axis_index("s")
    wid = s * NC + c
    pltpu.sync_copy(zeros_hbm.at[pl.ds(s * RPS, RPS)],
                    deg_sp.at[pl.ds(s * RPS, RPS)])
    pltpu.async_copy(dst_hbm.at[wid], idx_dst, sem).wait()
    for i in range(CHUNK // 16):
        ones_v[pl.ds(i * 16, 16)] = jnp.ones((16,), jnp.float32)
    plsc.subcore_barrier()

    def body(j, carry):
        pltpu.sync_copy(ones_v, deg_sp.at[idx_dst.at[j]], add=True)
        return carry

    lax.fori_loop(0, CPW, body, 0)
    plsc.subcore_barrier()
    pltpu.sync_copy(deg_sp.at[pl.ds(s * RPS, RPS)],
                    deg_hbm.at[c, pl.ds(s * RPS, RPS)])


_sc_deg = functools.partial(
    pl.kernel,
    out_type=jax.ShapeDtypeStruct((NC, NP), jnp.float32),
    mesh=_SC_MESH,
    scratch_types=[
        pltpu.VMEM((CPW, CHUNK), jnp.int32),
        pltpu.VMEM((CHUNK,), jnp.float32),
        pltpu.VMEM_SHARED((NP,), jnp.float32),
        pltpu.SemaphoreType.DMA,
    ],
    compiler_params=_SC_PARAMS,
)(_sc_deg_body)


def _sc_edge_body(src_hbm, dst_hbm, g_hbm, zeros_hbm, acc_hbm,
                  idx_src, idx_dst, rows, acc_sp, sem):
    c = lax.axis_index("c")
    s = lax.axis_index("s")
    wid = s * NC + c
    pltpu.sync_copy(zeros_hbm.at[pl.ds(s * RPS, RPS)],
                    acc_sp.at[pl.ds(s * RPS, RPS)])
    pltpu.async_copy(src_hbm.at[wid], idx_src, sem).wait()
    pltpu.async_copy(dst_hbm.at[wid], idx_dst, sem).wait()
    plsc.subcore_barrier()

    def body(j, carry):
        pltpu.async_copy(g_hbm.at[idx_src.at[j]], rows, sem).wait()
        pltpu.sync_copy(rows, acc_sp.at[idx_dst.at[j]], add=True)
        return carry

    lax.fori_loop(0, CPW, body, 0)
    plsc.subcore_barrier()
    pltpu.sync_copy(acc_sp.at[pl.ds(s * RPS, RPS)],
                    acc_hbm.at[c, pl.ds(s * RPS, RPS)])


_sc_edge = functools.partial(
    pl.kernel,
    out_type=jax.ShapeDtypeStruct((NC, NP, HH), jnp.float32),
    mesh=_SC_MESH,
    scratch_types=[
        pltpu.VMEM((CPW, CHUNK), jnp.int32),
        pltpu.VMEM((CPW, CHUNK), jnp.int32),
        pltpu.VMEM((CHUNK, HH), jnp.float32),
        pltpu.VMEM_SHARED((NP, HH), jnp.float32),
        pltpu.SemaphoreType.DMA,
    ],
    compiler_params=_SC_PARAMS,
)(_sc_edge_body)


def _tc_dinv_body(deg_ref, out_ref):
    d = deg_ref[0] + deg_ref[1] + 1.0
    out_ref[...] = lax.rsqrt(d)


def _tc_dinv(deg2d):
    return pl.pallas_call(
        _tc_dinv_body,
        out_shape=jax.ShapeDtypeStruct((NP // 128, 128), jnp.float32),
    )(deg2d)


def _tc_pre_body(x_ref, w_ref, dinv_ref, g_ref):
    h = jnp.dot(x_ref[...], w_ref[...], preferred_element_type=jnp.float32)
    g_ref[...] = dinv_ref[...] * h


def _tc_pre(xp, W, dinv32):
    blk = 1024
    return pl.pallas_call(
        _tc_pre_body,
        grid=(NP // blk,),
        in_specs=[
            pl.BlockSpec((blk, DI), lambda i: (i, 0)),
            pl.BlockSpec((DI, HH), lambda i: (0, 0)),
            pl.BlockSpec((blk, HH), lambda i: (i, 0)),
        ],
        out_specs=pl.BlockSpec((blk, HH), lambda i: (i, 0)),
        out_shape=jax.ShapeDtypeStruct((NP, HH), jnp.float32),
    )(xp, W, dinv32)


def _norm_relu(acc_ref, g_ref, dinv_ref, b_ref):
    s = acc_ref[0] + acc_ref[1] + g_ref[...]
    pre = dinv_ref[...] * s + b_ref[...]
    nrm = jnp.sqrt(jnp.sum(pre * pre, axis=1, keepdims=True))
    return jnp.maximum(pre / jnp.maximum(nrm, 1e-12), 0.0)


def _tc_post_body(acc_ref, g_ref, dinv_ref, b_ref, w_ref, gnext_ref):
    o = _norm_relu(acc_ref, g_ref, dinv_ref, b_ref)
    gnext_ref[...] = dinv_ref[...] * jnp.dot(
        o, w_ref[...], preferred_element_type=jnp.float32)


def _tc_post(acc, g, dinv32, b, Wn):
    blk = 1024
    return pl.pallas_call(
        _tc_post_body,
        grid=(NP // blk,),
        in_specs=[
            pl.BlockSpec((NC, blk, HH), lambda i: (0, i, 0)),
            pl.BlockSpec((blk, HH), lambda i: (i, 0)),
            pl.BlockSpec((blk, HH), lambda i: (i, 0)),
            pl.BlockSpec((1, HH), lambda i: (0, 0)),
            pl.BlockSpec((HH, HH), lambda i: (0, 0)),
        ],
        out_specs=pl.BlockSpec((blk, HH), lambda i: (i, 0)),
        out_shape=jax.ShapeDtypeStruct((NP, HH), jnp.float32),
    )(acc, g, dinv32, b, Wn)


def _tc_readout_body(acc_ref, g_ref, dinv_ref, b_ref, batch_ref,
                     linw_ref, linb_ref, out_ref, mx_s, sm_s, ct_s):
    i = pl.program_id(0)
    nblk = pl.num_programs(0)

    @pl.when(i == 0)
    def _init():
        mx_s[...] = jnp.zeros_like(mx_s)
        sm_s[...] = jnp.zeros_like(sm_s)
        ct_s[...] = jnp.zeros_like(ct_s)

    o = _norm_relu(acc_ref, g_ref, dinv_ref, b_ref)
    batch = batch_ref[...]
    lo = batch_ref[0, 0]
    hi = batch_ref[batch_ref.shape[0] - 1, 0]

    def body(gid, carry):
        m = batch == gid
        vals = jnp.where(m, o, 0.0)
        mx = jnp.max(vals, axis=0, keepdims=True)
        sm = jnp.sum(vals, axis=0, keepdims=True)
        ct = jnp.sum(jnp.where(m, 1.0, 0.0), axis=0, keepdims=True)
        mx_s[pl.ds(gid, 1), :] = jnp.maximum(mx_s[pl.ds(gid, 1), :], mx)
        sm_s[pl.ds(gid, 1), :] = sm_s[pl.ds(gid, 1), :] + sm
        ct_s[pl.ds(gid, 1), :] = ct_s[pl.ds(gid, 1), :] + ct
        return carry

    lax.fori_loop(lo, hi + 1, body, 0)

    @pl.when(i == nblk - 1)
    def _fin():
        mean = sm_s[0:GG, :] / jnp.maximum(ct_s[0:GG, :], 1.0)
        pooled = jnp.concatenate([mx_s[0:GG, :], mean], axis=1)
        out_ref[...] = jnp.dot(
            pooled, linw_ref[...], preferred_element_type=jnp.float32
        ) + linb_ref[...]


def _tc_readout(acc, g, dinv32, b, batchb, linW, linb):
    blk = 512
    return pl.pallas_call(
        _tc_readout_body,
        grid=(NP // blk,),
        in_specs=[
            pl.BlockSpec((NC, blk, HH), lambda i: (0, i, 0)),
            pl.BlockSpec((blk, HH), lambda i: (i, 0)),
            pl.BlockSpec((blk, HH), lambda i: (i, 0)),
            pl.BlockSpec((1, HH), lambda i: (0, 0)),
            pl.BlockSpec((blk, HH), lambda i: (i, 0)),
            pl.BlockSpec((2 * HH, CC), lambda i: (0, 0)),
            pl.BlockSpec((1, CC), lambda i: (0, 0)),
        ],
        out_specs=pl.BlockSpec((GG, CC), lambda i: (0, 0)),
        out_shape=jax.ShapeDtypeStruct((GG, CC), jnp.float32),
        scratch_shapes=[
            pltpu.VMEM((GG + 8, HH), jnp.float32),
            pltpu.VMEM((GG + 8, HH), jnp.float32),
            pltpu.VMEM((GG + 8, HH), jnp.float32),
        ],
    )(acc, g, dinv32, b, batchb, linW, linb)


def kernel(x, edge_index, batch, W1, b1, W2, b2, W3, b3, linW, linb):
    f32 = jnp.float32
    xp = jnp.zeros((NP, DI), f32).at[:NN].set(x)
    pad_ids = (NN + jnp.arange(EP - EE, dtype=jnp.int32) % (NP - NN))
    src = jnp.concatenate([edge_index[0], pad_ids]).reshape(NW, CPW, CHUNK)
    dst = jnp.concatenate([edge_index[1], pad_ids]).reshape(NW, CPW, CHUNK)
    batchp = jnp.full((NP,), GG, jnp.int32).at[:NN].set(batch)
    batchb = jnp.broadcast_to(batchp[:, None], (NP, HH))
    zeros_nh = jnp.zeros((NP, HH), f32)
    zeros_n = jnp.zeros((NP,), f32)

    deg2 = _sc_deg(dst, zeros_n)
    dinv2d = _tc_dinv(deg2.reshape(NC, NP // 128, 128))
    dinv32 = jnp.broadcast_to(dinv2d.reshape(NP, 1), (NP, HH))

    g1 = _tc_pre(xp, W1, dinv32)
    acc1 = _sc_edge(src, dst, g1, zeros_nh)
    g2 = _tc_post(acc1, g1, dinv32, b1.reshape(1, HH), W2)
    acc2 = _sc_edge(src, dst, g2, zeros_nh)
    g3 = _tc_post(acc2, g2, dinv32, b2.reshape(1, HH), W3)
    acc3 = _sc_edge(src, dst, g3, zeros_nh)
    return _tc_readout(acc3, g3, dinv32, b3.reshape(1, HH), batchb,
                       linW, linb.reshape(1, CC))


# speedup vs baseline: 44.0758x; 1.5657x over previous
import functools
import jax
import jax.numpy as jnp
from jax import lax
from jax.experimental import pallas as pl
from jax.experimental.pallas import tpu as pltpu
from jax.experimental.pallas import tpu_sc as plsc

NN = 10000
NP = 10240
EE = 320000
EP = 327680
DI = 128
HH = 32
CC = 10
GG = 64

NC = 2
NS = 16
NW = NC * NS
CHUNK = 128
CPW = EP // NW // CHUNK
RPS = NP // NS

_SC_MESH = plsc.VectorSubcoreMesh(core_axis_name="c", subcore_axis_name="s")
_SC_PARAMS = pltpu.CompilerParams(use_tc_tiling_on_sc=False)


def _sc_deg_body(dst_hbm, zeros_hbm, deg_hbm, idx_dst, ones_v, deg_sp, sem):
    c = lax.axis_index("c")
    s = lax.axis_index("s")
    wid = s * NC + c
    pltpu.sync_copy(zeros_hbm.at[pl.ds(s * RPS, RPS)],
                    deg_sp.at[pl.ds(s * RPS, RPS)])
    pltpu.async_copy(dst_hbm.at[wid], idx_dst, sem).wait()
    for i in range(CHUNK // 16):
        ones_v[pl.ds(i * 16, 16)] = jnp.ones((16,), jnp.float32)
    plsc.subcore_barrier()

    def body(j, carry):
        pltpu.sync_copy(ones_v, deg_sp.at[idx_dst.at[j]], add=True)
        return carry

    lax.fori_loop(0, CPW, body, 0)
    plsc.subcore_barrier()
    pltpu.sync_copy(deg_sp.at[pl.ds(s * RPS, RPS)],
                    deg_hbm.at[c, pl.ds(s * RPS, RPS)])


_sc_deg = functools.partial(
    pl.kernel,
    out_type=jax.ShapeDtypeStruct((NC, NP), jnp.float32),
    mesh=_SC_MESH,
    scratch_types=[
        pltpu.VMEM((CPW, CHUNK), jnp.int32),
        pltpu.VMEM((CHUNK,), jnp.float32),
        pltpu.VMEM_SHARED((NP,), jnp.float32),
        pltpu.SemaphoreType.DMA,
    ],
    compiler_params=_SC_PARAMS,
)(_sc_deg_body)


NBUF = 4


def _sc_edge_body(src_hbm, dst_hbm, g_hbm, zeros_hbm, acc_hbm,
                  idx_src, idx_dst, rows, acc_sp, gsems, ssems):
    c = lax.axis_index("c")
    s = lax.axis_index("s")
    wid = s * NC + c
    pltpu.sync_copy(zeros_hbm.at[pl.ds(s * RPS, RPS)],
                    acc_sp.at[pl.ds(s * RPS, RPS)])
    pltpu.async_copy(src_hbm.at[wid], idx_src, gsems.at[0]).wait()
    pltpu.async_copy(dst_hbm.at[wid], idx_dst, gsems.at[0]).wait()
    plsc.subcore_barrier()
    for b in range(NBUF):
        pltpu.async_copy(g_hbm.at[idx_src.at[b]], rows.at[b], gsems.at[b])

    def body(i, carry):
        for b in range(NBUF):
            j = i * NBUF + b
            pltpu.make_async_copy(g_hbm.at[idx_src.at[0]], rows.at[b],
                                  gsems.at[b]).wait()
            pltpu.async_copy(rows.at[b], acc_sp.at[idx_dst.at[j]],
                             ssems.at[b], add=True)
        for b in range(NBUF):
            j = i * NBUF + b
            pltpu.make_async_copy(rows.at[b], acc_sp.at[pl.ds(0, CHUNK)],
                                  ssems.at[b]).wait()

            @pl.when(j + NBUF < CPW)
            def _refill():
                pltpu.async_copy(g_hbm.at[idx_src.at[j + NBUF]], rows.at[b],
                                 gsems.at[b])
        return carry

    lax.fori_loop(0, CPW // NBUF, body, 0)
    plsc.subcore_barrier()
    pltpu.sync_copy(acc_sp.at[pl.ds(s * RPS, RPS)],
                    acc_hbm.at[c, pl.ds(s * RPS, RPS)])


_sc_edge = functools.partial(
    pl.kernel,
    out_type=jax.ShapeDtypeStruct((NC, NP, HH), jnp.float32),
    mesh=_SC_MESH,
    scratch_types=[
        pltpu.VMEM((CPW, CHUNK), jnp.int32),
        pltpu.VMEM((CPW, CHUNK), jnp.int32),
        pltpu.VMEM((NBUF, CHUNK, HH), jnp.float32),
        pltpu.VMEM_SHARED((NP, HH), jnp.float32),
        pltpu.SemaphoreType.DMA((NBUF,)),
        pltpu.SemaphoreType.DMA((NBUF,)),
    ],
    compiler_params=_SC_PARAMS,
)(_sc_edge_body)


def _tc_dinv_body(deg_ref, out_ref):
    d = deg_ref[0] + deg_ref[1] + 1.0
    out_ref[...] = lax.rsqrt(d)


def _tc_dinv(deg2d):
    return pl.pallas_call(
        _tc_dinv_body,
        out_shape=jax.ShapeDtypeStruct((NP // 128, 128), jnp.float32),
    )(deg2d)


def _tc_pre_body(x_ref, w_ref, dinv_ref, g_ref):
    h = jnp.dot(x_ref[...], w_ref[...], preferred_element_type=jnp.float32)
    g_ref[...] = dinv_ref[...] * h


def _tc_pre(xp, W, dinv32):
    blk = 1024
    return pl.pallas_call(
        _tc_pre_body,
        grid=(NP // blk,),
        in_specs=[
            pl.BlockSpec((blk, DI), lambda i: (i, 0)),
            pl.BlockSpec((DI, HH), lambda i: (0, 0)),
            pl.BlockSpec((blk, HH), lambda i: (i, 0)),
        ],
        out_specs=pl.BlockSpec((blk, HH), lambda i: (i, 0)),
        out_shape=jax.ShapeDtypeStruct((NP, HH), jnp.float32),
    )(xp, W, dinv32)


def _norm_relu(acc_ref, g_ref, dinv_ref, b_ref):
    s = acc_ref[0] + acc_ref[1] + g_ref[...]
    pre = dinv_ref[...] * s + b_ref[...]
    nrm = jnp.sqrt(jnp.sum(pre * pre, axis=1, keepdims=True))
    return jnp.maximum(pre / jnp.maximum(nrm, 1e-12), 0.0)


def _tc_post_body(acc_ref, g_ref, dinv_ref, b_ref, w_ref, gnext_ref):
    o = _norm_relu(acc_ref, g_ref, dinv_ref, b_ref)
    gnext_ref[...] = dinv_ref[...] * jnp.dot(
        o, w_ref[...], preferred_element_type=jnp.float32)


def _tc_post(acc, g, dinv32, b, Wn):
    blk = 1024
    return pl.pallas_call(
        _tc_post_body,
        grid=(NP // blk,),
        in_specs=[
            pl.BlockSpec((NC, blk, HH), lambda i: (0, i, 0)),
            pl.BlockSpec((blk, HH), lambda i: (i, 0)),
            pl.BlockSpec((blk, HH), lambda i: (i, 0)),
            pl.BlockSpec((1, HH), lambda i: (0, 0)),
            pl.BlockSpec((HH, HH), lambda i: (0, 0)),
        ],
        out_specs=pl.BlockSpec((blk, HH), lambda i: (i, 0)),
        out_shape=jax.ShapeDtypeStruct((NP, HH), jnp.float32),
    )(acc, g, dinv32, b, Wn)


def _tc_readout_body(acc_ref, g_ref, dinv_ref, b_ref, batch_ref,
                     linw_ref, linb_ref, out_ref, mx_s, sm_s, ct_s):
    i = pl.program_id(0)
    nblk = pl.num_programs(0)

    @pl.when(i == 0)
    def _init():
        mx_s[...] = jnp.zeros_like(mx_s)
        sm_s[...] = jnp.zeros_like(sm_s)
        ct_s[...] = jnp.zeros_like(ct_s)

    o = _norm_relu(acc_ref, g_ref, dinv_ref, b_ref)
    batch = batch_ref[...]
    lo = batch_ref[0, 0]
    hi = batch_ref[batch_ref.shape[0] - 1, 0]

    def body(gid, carry):
        m = batch == gid
        vals = jnp.where(m, o, 0.0)
        mx = jnp.max(vals, axis=0, keepdims=True)
        sm = jnp.sum(vals, axis=0, keepdims=True)
        ct = jnp.sum(jnp.where(m, 1.0, 0.0), axis=0, keepdims=True)
        mx_s[pl.ds(gid, 1), :] = jnp.maximum(mx_s[pl.ds(gid, 1), :], mx)
        sm_s[pl.ds(gid, 1), :] = sm_s[pl.ds(gid, 1), :] + sm
        ct_s[pl.ds(gid, 1), :] = ct_s[pl.ds(gid, 1), :] + ct
        return carry

    lax.fori_loop(lo, hi + 1, body, 0)

    @pl.when(i == nblk - 1)
    def _fin():
        mean = sm_s[0:GG, :] / jnp.maximum(ct_s[0:GG, :], 1.0)
        pooled = jnp.concatenate([mx_s[0:GG, :], mean], axis=1)
        out_ref[...] = jnp.dot(
            pooled, linw_ref[...], preferred_element_type=jnp.float32
        ) + linb_ref[...]


def _tc_readout(acc, g, dinv32, b, batchb, linW, linb):
    blk = 512
    return pl.pallas_call(
        _tc_readout_body,
        grid=(NP // blk,),
        in_specs=[
            pl.BlockSpec((NC, blk, HH), lambda i: (0, i, 0)),
            pl.BlockSpec((blk, HH), lambda i: (i, 0)),
            pl.BlockSpec((blk, HH), lambda i: (i, 0)),
            pl.BlockSpec((1, HH), lambda i: (0, 0)),
            pl.BlockSpec((blk, HH), lambda i: (i, 0)),
            pl.BlockSpec((2 * HH, CC), lambda i: (0, 0)),
            pl.BlockSpec((1, CC), lambda i: (0, 0)),
        ],
        out_specs=pl.BlockSpec((GG, CC), lambda i: (0, 0)),
        out_shape=jax.ShapeDtypeStruct((GG, CC), jnp.float32),
        scratch_shapes=[
            pltpu.VMEM((GG + 8, HH), jnp.float32),
            pltpu.VMEM((GG + 8, HH), jnp.float32),
            pltpu.VMEM((GG + 8, HH), jnp.float32),
        ],
    )(acc, g, dinv32, b, batchb, linW, linb)


def kernel(x, edge_index, batch, W1, b1, W2, b2, W3, b3, linW, linb):
    f32 = jnp.float32
    xp = jnp.zeros((NP, DI), f32).at[:NN].set(x)
    pad_ids = (NN + jnp.arange(EP - EE, dtype=jnp.int32) % (NP - NN))
    src = jnp.concatenate([edge_index[0], pad_ids]).reshape(NW, CPW, CHUNK)
    dst = jnp.concatenate([edge_index[1], pad_ids]).reshape(NW, CPW, CHUNK)
    batchp = jnp.full((NP,), GG, jnp.int32).at[:NN].set(batch)
    batchb = jnp.broadcast_to(batchp[:, None], (NP, HH))
    zeros_nh = jnp.zeros((NP, HH), f32)
    zeros_n = jnp.zeros((NP,), f32)

    deg2 = _sc_deg(dst, zeros_n)
    dinv2d = _tc_dinv(deg2.reshape(NC, NP // 128, 128))
    dinv32 = jnp.broadcast_to(dinv2d.reshape(NP, 1), (NP, HH))

    g1 = _tc_pre(xp, W1, dinv32)
    acc1 = _sc_edge(src, dst, g1, zeros_nh)
    g2 = _tc_post(acc1, g1, dinv32, b1.reshape(1, HH), W2)
    acc2 = _sc_edge(src, dst, g2, zeros_nh)
    g3 = _tc_post(acc2, g2, dinv32, b2.reshape(1, HH), W3)
    acc3 = _sc_edge(src, dst, g3, zeros_nh)
    return _tc_readout(acc3, g3, dinv32, b3.reshape(1, HH), batchb,
                       linW, linb.reshape(1, CC))
